# Initial kernel scaffold; baseline (speedup 1.0000x reference)
import jax
import jax.numpy as jnp
from jax.experimental import pallas as pl


def kernel(x, edge_index_0, edge_index_1, W1_0, b1_0, W1_1, b1_1, W2_0, b2_0, W2_1, b2_1):
    raise NotImplementedError("write your pallas kernel here")


# speedup vs baseline: 22.6005x; 22.6005x over previous
import functools

import jax
import jax.numpy as jnp
from jax import lax
from jax.experimental import pallas as pl
from jax.experimental.pallas import tpu as pltpu
from jax.experimental.pallas import tpu_sc as plsc

N = 10000
E = 320000
D = 128
NC = 2
NS = 16
EPT = E // NS
CHUNK = 128
NCH = (EPT + CHUNK - 1) // CHUNK
EPAD = NCH * CHUNK
ROWS_PT = N // NS
RBLK = 1000
GRID = N // RBLK
HPAD = ((N + 1 + 15) // 16) * 16


_sc_mesh = plsc.VectorSubcoreMesh(core_axis_name="c", subcore_axis_name="s")


@functools.partial(
    pl.kernel,
    out_type=jax.ShapeDtypeStruct((NC, NS, HPAD), jnp.float32),
    mesh=_sc_mesh,
    scratch_types=[
        pltpu.VMEM((EPAD,), jnp.int32),
        pltpu.VMEM((HPAD,), jnp.float32),
    ],
    compiler_params=pltpu.CompilerParams(needs_layout_passes=False),
)
def _sc_degree(cols_hbm, hist_hbm, col_v, hist_v):
    c = lax.axis_index("c")
    s = lax.axis_index("s")
    pltpu.sync_copy(cols_hbm.at[c, s], col_v)

    zeros16 = jnp.zeros((16,), jnp.float32)

    def zbody(i, _):
        hist_v[pl.ds(i * 16, 16)] = zeros16
        return ()

    lax.fori_loop(0, HPAD // 16, zbody, (), unroll=8)

    ones16 = jnp.ones((16,), jnp.float32)

    def hbody(i, _):
        idx = col_v[pl.ds(i * 16, 16)]
        plsc.addupdate_scatter(hist_v, [idx], ones16)
        return ()

    lax.fori_loop(0, EPAD // 16, hbody, (), unroll=8)
    pltpu.sync_copy(hist_v, hist_hbm.at[c, s])


@functools.partial(
    pl.kernel,
    out_type=jax.ShapeDtypeStruct((NC, N, D), jnp.float32),
    mesh=_sc_mesh,
    scratch_types=[
        pltpu.VMEM((2, CHUNK), jnp.int32),
        pltpu.VMEM((2, CHUNK), jnp.int32),
        pltpu.VMEM((CHUNK, D), jnp.float32),
        pltpu.VMEM((CHUNK, D), jnp.float32),
        pltpu.VMEM_SHARED((N + 8, D), jnp.float32),
        pltpu.SemaphoreType.DMA,
        pltpu.SemaphoreType.DMA,
        pltpu.SemaphoreType.DMA,
        pltpu.SemaphoreType.DMA,
    ],
)
def _sc_scatter(g_hbm, idx_hbm, acc_hbm,
                ibuf0, ibuf1, buf0, buf1, acc_sh, semi0, semi1, semg0, semg1):
    c = lax.axis_index("c")
    s = lax.axis_index("s")
    gflat = g_hbm.at[c]
    myidx = idx_hbm.at[c, s]

    @pl.when(s < NS - 1)
    def _():
        pltpu.sync_copy(gflat.at[pl.ds(s * 640, 640)],
                        acc_sh.at[pl.ds(s * 640, 640)])

    @pl.when(s == NS - 1)
    def _():
        pltpu.sync_copy(gflat.at[pl.ds(9600, 400)],
                        acc_sh.at[pl.ds(9600, 400)])

    plsc.subcore_barrier()

    pltpu.sync_copy(myidx.at[0], ibuf0)
    pltpu.async_copy(myidx.at[1], ibuf1, semi1)
    pltpu.async_copy(gflat.at[ibuf0.at[0]], buf0, semg0)

    def body(jj, _):
        j0 = 2 * jj
        pltpu.make_async_copy(gflat.at[ibuf0.at[0]], buf0, semg0).wait()
        pltpu.make_async_copy(myidx.at[j0 + 1], ibuf1, semi1).wait()
        pltpu.async_copy(gflat.at[ibuf1.at[0]], buf1, semg1)
        pltpu.sync_copy(buf0, acc_sh.at[ibuf0.at[1]], add=True)
        pltpu.async_copy(myidx.at[j0 + 2], ibuf0, semi0)
        pltpu.make_async_copy(gflat.at[ibuf1.at[0]], buf1, semg1).wait()
        pltpu.sync_copy(buf1, acc_sh.at[ibuf1.at[1]], add=True)
        pltpu.make_async_copy(myidx.at[j0 + 2], ibuf0, semi0).wait()
        pltpu.async_copy(gflat.at[ibuf0.at[0]], buf0, semg0)
        pltpu.async_copy(myidx.at[j0 + 3], ibuf1, semi1)
        return ()

    lax.fori_loop(0, (NCH - 3) // 2, body, ())
    pltpu.make_async_copy(gflat.at[ibuf0.at[0]], buf0, semg0).wait()
    pltpu.make_async_copy(myidx.at[NCH - 2], ibuf1, semi1).wait()
    pltpu.async_copy(gflat.at[ibuf1.at[0]], buf1, semg1)
    pltpu.sync_copy(buf0, acc_sh.at[ibuf0.at[1]], add=True)
    pltpu.async_copy(myidx.at[NCH - 1], ibuf0, semi0)
    pltpu.make_async_copy(gflat.at[ibuf1.at[0]], buf1, semg1).wait()
    pltpu.sync_copy(buf1, acc_sh.at[ibuf1.at[1]], add=True)
    pltpu.make_async_copy(myidx.at[NCH - 1], ibuf0, semi0).wait()
    pltpu.async_copy(gflat.at[ibuf0.at[0]], buf0, semg0)
    pltpu.make_async_copy(gflat.at[ibuf0.at[0]], buf0, semg0).wait()
    pltpu.sync_copy(buf0, acc_sh.at[ibuf0.at[1]], add=True)

    plsc.subcore_barrier()

    @pl.when(s < NS - 1)
    def _():
        pltpu.sync_copy(acc_sh.at[pl.ds(s * 640, 640)],
                        acc_hbm.at[c].at[pl.ds(s * 640, 640)])

    @pl.when(s == NS - 1)
    def _():
        pltpu.sync_copy(acc_sh.at[pl.ds(9600, 400)],
                        acc_hbm.at[c].at[pl.ds(9600, 400)])


def _tc_dinv_body(hist_ref, dinv_ref):
    deg = jnp.sum(hist_ref[...], axis=1) + 1.0
    dinv_ref[...] = lax.rsqrt(deg)[:, :N, None]


_tc_dinv = pl.pallas_call(
    _tc_dinv_body,
    out_shape=jax.ShapeDtypeStruct((NC, N, 1), jnp.float32),
)


def _tc_layer1_body(x_ref, w0_ref, w1_ref, dinv_ref, g_ref):
    dinv = dinv_ref[...]
    xb = x_ref[...]
    h0 = jnp.dot(xb, w0_ref[...], preferred_element_type=jnp.float32)
    h1 = jnp.dot(xb, w1_ref[...], preferred_element_type=jnp.float32)
    g_ref[0] = h0 * dinv[0]
    g_ref[1] = h1 * dinv[1]


def _tc_layer2_body(acc_ref, dinv_ref, b1_ref, w0_ref, w1_ref, g_ref):
    dinv = dinv_ref[...]
    h = jax.nn.relu(acc_ref[0] * dinv[0] + b1_ref[0]
                    + acc_ref[1] * dinv[1] + b1_ref[1])
    h0 = jnp.dot(h, w0_ref[...], preferred_element_type=jnp.float32)
    h1 = jnp.dot(h, w1_ref[...], preferred_element_type=jnp.float32)
    g_ref[0] = h0 * dinv[0]
    g_ref[1] = h1 * dinv[1]


def _tc_final_body(acc_ref, dinv_ref, b2_ref, out_ref):
    dinv = dinv_ref[...]
    out_ref[...] = (acc_ref[0] * dinv[0] + b2_ref[0]
                    + acc_ref[1] * dinv[1] + b2_ref[1])


_w_spec = pl.BlockSpec((D, D), lambda i: (0, 0))
_b_spec = pl.BlockSpec((NC, 1, D), lambda i: (0, 0, 0))
_g_spec = pl.BlockSpec((NC, RBLK, D), lambda i: (0, i, 0))
_dinv_spec = pl.BlockSpec((NC, RBLK, 1), lambda i: (0, i, 0))
_x_spec = pl.BlockSpec((RBLK, D), lambda i: (i, 0))

_tc_layer1 = pl.pallas_call(
    _tc_layer1_body,
    grid=(GRID,),
    in_specs=[_x_spec, _w_spec, _w_spec, _dinv_spec],
    out_specs=_g_spec,
    out_shape=jax.ShapeDtypeStruct((NC, N, D), jnp.float32),
)

_tc_layer2 = pl.pallas_call(
    _tc_layer2_body,
    grid=(GRID,),
    in_specs=[_g_spec, _dinv_spec, _b_spec, _w_spec, _w_spec],
    out_specs=_g_spec,
    out_shape=jax.ShapeDtypeStruct((NC, N, D), jnp.float32),
)

_tc_final = pl.pallas_call(
    _tc_final_body,
    grid=(GRID,),
    in_specs=[_g_spec, _dinv_spec, _b_spec],
    out_specs=_x_spec,
    out_shape=jax.ShapeDtypeStruct((N, D), jnp.float32),
)


def _prep_indices(ei):
    r = ei[0].astype(jnp.int32).reshape(NS, EPT)
    c = ei[1].astype(jnp.int32).reshape(NS, EPT)
    pad = ((0, 0), (0, EPAD - EPT))
    r = jnp.pad(r, pad, constant_values=0).reshape(NS, NCH, CHUNK)
    c = jnp.pad(c, pad, constant_values=N).reshape(NS, NCH, CHUNK)
    return jnp.stack([r, c], axis=2), c.reshape(NS, EPAD)


@jax.jit
def kernel(x, edge_index_0, edge_index_1,
           W1_0, b1_0, W1_1, b1_1, W2_0, b2_0, W2_1, b2_1):
    i0, c0 = _prep_indices(edge_index_0)
    i1, c1 = _prep_indices(edge_index_1)
    idx = jnp.stack([i0, i1])
    cols_flat = jnp.stack([c0, c1])

    hist = _sc_degree(cols_flat)
    b1 = jnp.stack([b1_0, b1_1]).reshape(NC, 1, D)
    b2 = jnp.stack([b2_0, b2_1]).reshape(NC, 1, D)

    dinv = _tc_dinv(hist)
    g1 = _tc_layer1(x, W1_0, W1_1, dinv)
    acc1 = _sc_scatter(g1, idx)
    g2 = _tc_layer2(acc1, dinv, b1, W2_0, W2_1)
    acc2 = _sc_scatter(g2, idx)
    return _tc_final(acc2, dinv, b2)
